# baseline (speedup 1.0000x reference)
import functools

import jax
import jax.numpy as jnp
from jax import lax
from jax.experimental import pallas as pl
from jax.experimental.pallas import tpu as pltpu
from jax.experimental.pallas import tpu_sc as plsc

N = 50000
E = 800000
H = 32
R = 3

NC = 2
NS = 16
NW = NC * NS
G = 128
BLK = 1280
NGRP = BLK // G
NBLKS = E // BLK
BPW = -(-NBLKS // NW)
E2 = 819200
BE = 8192

NRP = 153600
SLOT_PER_SUB = NRP // NS
AGG_ROWS = 50176
AGG_PER_SUB = AGG_ROWS // NS
WAVE = 5
WAVE_E = WAVE * G
CAP_R = 26880
NWAVES = CAP_R // WAVE_E
BLKP = BLK + 16
NBLKS2 = E2 // BLK
BPW2 = NBLKS2 // NW

_mesh = plsc.VectorSubcoreMesh(core_axis_name="c", subcore_axis_name="s")
_f32 = jnp.float32
_i32 = jnp.int32


@functools.partial(
    pl.kernel,
    out_type=[
        jax.ShapeDtypeStruct((E2,), _f32),
        jax.ShapeDtypeStruct((E2,), _f32),
    ],
    mesh=_mesh,
    scratch_types=[
        pltpu.VMEM((BLK,), _i32),
        pltpu.VMEM((BLK,), _i32),
        pltpu.VMEM((BLK,), _f32),
        pltpu.VMEM((BLK,), _f32),
        pltpu.SemaphoreType.DMA,
    ],
)
def _phase_a1(x_hbm, row_hbm, col_hbm, xr_hbm, xc_hbm,
              rowb, colb, xrb, xcb, semg):
    cid = lax.axis_index("c")
    sid = lax.axis_index("s")
    wid = cid * NS + sid

    @pl.loop(0, BPW)
    def _block(bi):
        cidx = wid + bi * NW

        @pl.when(cidx < NBLKS)
        def _():
            ebase = cidx * BLK
            cl = pltpu.async_copy(row_hbm.at[pl.ds(ebase, BLK)], rowb, semg)
            cl2 = pltpu.async_copy(col_hbm.at[pl.ds(ebase, BLK)], colb, semg)
            cl.wait()
            cl2.wait()
            gs = []
            for i in range(NGRP):
                sl = pl.ds(i * G, G)
                gs.append(pltpu.async_copy(
                    x_hbm.at[rowb.at[sl]], xrb.at[sl], semg))
                gs.append(pltpu.async_copy(
                    x_hbm.at[colb.at[sl]], xcb.at[sl], semg))
            for c in gs:
                c.wait()
            st = [pltpu.async_copy(xrb, xr_hbm.at[pl.ds(ebase, BLK)], semg),
                  pltpu.async_copy(xcb, xc_hbm.at[pl.ds(ebase, BLK)], semg)]
            for c in st:
                c.wait()


def _phase_a2_body(xr_ref, xc_ref, wt_ref, bt_ref, w1_ref, b1_ref,
                   w2_ref, b2_ref, et_ref):
    xr = xr_ref[...]
    xc = xc_ref[...]
    wt = wt_ref[...]
    bt = bt_ref[...]
    efr = wt * xr[None, :] + bt
    efc = wt * xc[None, :] + bt
    ef = jnp.transpose(jnp.concatenate([efr, efc], axis=0))
    hid = jnp.maximum(jnp.dot(ef, w1_ref[...]) + b1_ref[...][None, :], 0.0)
    logits = jnp.dot(hid, w2_ref[...])
    lt = jnp.transpose(logits)
    b2 = b2_ref[...]
    l0 = lt[0] + b2[0]
    l1 = lt[1] + b2[1]
    l2 = lt[2] + b2[2]
    m = jnp.maximum(jnp.maximum(l0, l1), l2)
    u0 = jnp.exp(l0 - m)
    u1 = jnp.exp(l1 - m)
    u2 = jnp.exp(l2 - m)
    s = u0 + u1 + u2
    p0 = u0 / s
    p1 = u1 / s
    p2 = u2 / s
    et = jnp.zeros_like(et_ref[...])
    best = p0
    m1 = p1 > best
    et = jnp.where(m1, 1, et)
    best = jnp.where(m1, p1, best)
    et = jnp.where(p2 > best, 2, et)
    et_ref[...] = et


def _phase_a2(xr, xc, w, bemb, rp_W1, rp_b1, rp_W2, rp_b2):
    full = lambda shp: pl.BlockSpec(shp, lambda i: tuple(0 for _ in shp))
    wt = jnp.broadcast_to(w[:, None], (H, BE))
    bt = jnp.broadcast_to(bemb[:, None], (H, BE))
    return pl.pallas_call(
        _phase_a2_body,
        grid=(E2 // BE,),
        in_specs=[
            pl.BlockSpec((BE,), lambda i: (i,)),
            pl.BlockSpec((BE,), lambda i: (i,)),
            full((H, BE)), full((H, BE)), full((2 * H, H // 2)), full((H // 2,)),
            full((H // 2, R)), full((R,)),
        ],
        out_specs=pl.BlockSpec((BE,), lambda i: (i,)),
        out_shape=jax.ShapeDtypeStruct((E2,), _i32),
    )(xr, xc, wt, bt, rp_W1, rp_b1, rp_W2, rp_b2)


@functools.partial(
    pl.kernel,
    out_type=[
        jax.ShapeDtypeStruct((NC, NRP), _f32),
        jax.ShapeDtypeStruct((NC, NRP), _f32),
        jax.ShapeDtypeStruct((NW * R * CAP_R,), _i32),
        jax.ShapeDtypeStruct((NW * R * CAP_R,), _i32),
        jax.ShapeDtypeStruct((NW, 16), _i32),
    ],
    mesh=_mesh,
    scratch_types=[
        pltpu.VMEM((BLK,), _i32),
        pltpu.VMEM((BLK,), _i32),
        pltpu.VMEM((BLK,), _f32),
        pltpu.VMEM((BLK,), _i32),
        pltpu.VMEM((NGRP, G), _i32),
        pltpu.VMEM((BLK,), _f32),
        [pltpu.VMEM((BLKP,), _i32) for _ in range(R)],
        [pltpu.VMEM((BLKP,), _i32) for _ in range(R)],
        pltpu.VMEM((WAVE_E,), _i32),
        pltpu.VMEM((WAVE_E,), _i32),
        pltpu.VMEM((16,), _i32),
        pltpu.VMEM_SHARED((NRP,), _f32),
        pltpu.VMEM_SHARED((NRP,), _f32),
        pltpu.SemaphoreType.DMA,
        pltpu.SemaphoreType.DMA,
    ],
    compiler_params=pltpu.CompilerParams(needs_layout_passes=False),
)
def _phase_a3(col_hbm, et_hbm, xr_hbm, row_hbm, ones_hbm, zeros_hbm,
              padrow_hbm, padcol_hbm,
              sum_hbm, cnt_hbm, crow_hbm, ccol_hbm, cnts_hbm,
              colb, etb, xrb, rowb, didx2, onesb, crow_blk, ccol_blk,
              padr, padc, cbuf, SUM, CNT, semg, sems):
    cid = lax.axis_index("c")
    sid = lax.axis_index("s")
    wid = cid * NS + sid

    pltpu.sync_copy(ones_hbm, onesb)
    pltpu.sync_copy(padrow_hbm, padr)
    pltpu.sync_copy(padcol_hbm, padc)
    base0 = sid * SLOT_PER_SUB
    pltpu.sync_copy(zeros_hbm, SUM.at[pl.ds(base0, SLOT_PER_SUB)])
    pltpu.sync_copy(zeros_hbm, CNT.at[pl.ds(base0, SLOT_PER_SUB)])
    plsc.subcore_barrier()

    rbase = [(wid * R + r) * CAP_R for r in range(R)]

    def _block(bi, ns):
        cidx = wid + bi * NW
        ebase = cidx * BLK
        ls = [pltpu.async_copy(col_hbm.at[pl.ds(ebase, BLK)], colb, semg),
              pltpu.async_copy(et_hbm.at[pl.ds(ebase, BLK)], etb, semg),
              pltpu.async_copy(xr_hbm.at[pl.ds(ebase, BLK)], xrb, semg),
              pltpu.async_copy(row_hbm.at[pl.ds(ebase, BLK)], rowb, semg)]
        for c in ls:
            c.wait()

        def _grp(gi, cs):
            def _sixteen(jj, cs):
                j = gi * G + jj * 16
                sj = pl.ds(j, 16)
                et = etb[sj]
                cc = colb[sj]
                rr = rowb[sj]
                didx2[gi, pl.ds(jj * 16, 16)] = cc * R + et
                out = []
                for r in range(R):
                    m = et == r
                    plsc.store_compressed(crow_blk[r].at[pl.ds(cs[r], 16)],
                                          rr, mask=m)
                    plsc.store_compressed(ccol_blk[r].at[pl.ds(cs[r], 16)],
                                          cc, mask=m)
                    pc = plsc.all_reduce_population_count(m)
                    out.append(cs[r] + jnp.max(pc))
                return tuple(out)
            return lax.fori_loop(0, G // 16, _sixteen, cs)

        cs = lax.fori_loop(0, NGRP, _grp, (jnp.int32(0),) * R)

        lane16 = jax.lax.iota(_i32, 16)
        for r in range(R):
            crow_blk[r][pl.ds(cs[r], 16)] = jnp.zeros((16,), _i32)
            ccol_blk[r][pl.ds(cs[r], 16)] = N + lane16

        ss = []
        for i in range(NGRP):
            sl = pl.ds(i * G, G)
            ss.append(pltpu.async_copy(
                xrb.at[sl], SUM.at[didx2.at[i]], sems, add=True))
            ss.append(pltpu.async_copy(
                onesb.at[sl], CNT.at[didx2.at[i]], sems, add=True))
        ap = []
        for r in range(R):
            dst = pl.ds(pl.multiple_of(rbase[r] + ns[r], 16), BLKP)
            ap.append(pltpu.async_copy(crow_blk[r], crow_hbm.at[dst], semg))
            ap.append(pltpu.async_copy(ccol_blk[r], ccol_hbm.at[dst], semg))
        for c in ss + ap:
            c.wait()
        return tuple(ns[r] + ((cs[r] + 15) & ~15) for r in range(R))

    ns = lax.fori_loop(0, BPW2, _block, (jnp.int32(0),) * R)

    pd = []
    for r in range(R):
        dst = pl.ds(pl.multiple_of(rbase[r] + ns[r], 16), WAVE_E)
        pd.append(pltpu.async_copy(padr, crow_hbm.at[dst], semg))
        pd.append(pltpu.async_copy(padc, ccol_hbm.at[dst], semg))
    for c in pd:
        c.wait()
    lane = jax.lax.iota(_i32, 16)
    cvec = jnp.where(lane == 0, ns[0],
                     jnp.where(lane == 1, ns[1],
                               jnp.where(lane == 2, ns[2], 0)))
    cbuf[...] = cvec
    pltpu.sync_copy(cbuf, cnts_hbm.at[wid])

    plsc.subcore_barrier()
    sl = pl.ds(base0, SLOT_PER_SUB)
    pltpu.sync_copy(SUM.at[sl], sum_hbm.at[cid, sl])
    pltpu.sync_copy(CNT.at[sl], cnt_hbm.at[cid, sl])


@functools.partial(
    pl.kernel,
    out_type=jax.ShapeDtypeStruct((NC, R, AGG_ROWS, H), _f32),
    mesh=_mesh,
    scratch_types=[
        pltpu.VMEM((WAVE, G), _i32),
        pltpu.VMEM((WAVE, G), _i32),
        pltpu.VMEM((WAVE, G, H), _f32),
        pltpu.VMEM((16,), _i32),
        pltpu.VMEM_SHARED((AGG_ROWS, H), _f32),
        pltpu.SemaphoreType.DMA,
        pltpu.SemaphoreType.DMA,
    ],
    compiler_params=pltpu.CompilerParams(use_tc_tiling_on_sc=False),
)
def _phase_c(hmid_hbm, crow_hbm, ccol_hbm, cnts_hbm, zeros_hbm, agg_hbm,
             rowv, tgt2, grows, cbuf, AGG, semg, sems):
    cid = lax.axis_index("c")
    sid = lax.axis_index("s")
    wid = cid * NS + sid
    rbase = sid * AGG_PER_SUB

    pltpu.sync_copy(cnts_hbm.at[wid], cbuf)
    cv = cbuf[...]

    for r in range(R):
        pltpu.sync_copy(zeros_hbm, AGG.at[pl.ds(rbase, AGG_PER_SUB), :])
        plsc.subcore_barrier()
        nr = cv[r]

        @pl.loop(0, NWAVES)
        def _wave(wi):
            @pl.when(wi * WAVE_E < nr)
            def _():
                ls = [pltpu.async_copy(crow_hbm.at[wid, r, wi], rowv, semg),
                      pltpu.async_copy(ccol_hbm.at[wid, r, wi], tgt2, semg)]
                for c in ls:
                    c.wait()
                gs = [pltpu.async_copy(
                        hmid_hbm.at[rowv.at[i]], grows.at[i], semg)
                      for i in range(WAVE)]
                for c in gs:
                    c.wait()
                ss = [pltpu.async_copy(
                        grows.at[i], AGG.at[tgt2.at[i]], sems, add=True)
                      for i in range(WAVE)]
                for c in ss:
                    c.wait()

        plsc.subcore_barrier()
        pltpu.sync_copy(AGG.at[pl.ds(rbase, AGG_PER_SUB), :],
                        agg_hbm.at[cid, r, pl.ds(rbase, AGG_PER_SUB), :])
        plsc.subcore_barrier()


BN = 2000


def _phase_b_body(x_ref, s_ref, c_ref, u_ref, v_ref, g1_ref, h0_ref,
                  b1_ref, g_ref, bb_ref, w_ref, be_ref,
                  hmid_ref, cinv_ref):
    xb = x_ref[...]
    s = s_ref[0] + s_ref[1]
    c = c_ref[0] + c_ref[1]
    cc = jnp.maximum(c, 1.0)
    ci = 1.0 / cc
    msg = jnp.dot(s * ci, u_ref[...], preferred_element_type=_f32)
    msg = msg + jnp.dot(c * ci, v_ref[...], preferred_element_type=_f32)
    pre = xb * g1_ref[...][None, :] + h0_ref[...][None, :] + b1_ref[...][None, :] + msg
    m = jnp.mean(pre, axis=-1, keepdims=True)
    var = jnp.mean((pre - m) ** 2, axis=-1, keepdims=True)
    y = (pre - m) / jnp.sqrt(var + 1e-5) * g_ref[...][None, :] + bb_ref[...][None, :]
    h = xb * w_ref[...][None, :] + be_ref[...][None, :]
    hmid_ref[...] = jnp.maximum(y, 0.0) + h
    cinv_ref[...] = ci


def _phase_b(x, s_nr, c_nr, U, V, g1, h0, bias1, ln1_g, ln1_b, w, bemb):
    full = lambda shp: pl.BlockSpec(shp, lambda i: tuple(0 for _ in shp))
    return pl.pallas_call(
        _phase_b_body,
        grid=(N // BN,),
        in_specs=[
            pl.BlockSpec((BN, 1), lambda i: (i, 0)),
            pl.BlockSpec((NC, BN, R), lambda i: (0, i, 0)),
            pl.BlockSpec((NC, BN, R), lambda i: (0, i, 0)),
            full((R, H)), full((R, H)), full((H,)), full((H,)),
            full((H,)), full((H,)), full((H,)), full((H,)), full((H,)),
        ],
        out_specs=[
            pl.BlockSpec((BN, H), lambda i: (i, 0)),
            pl.BlockSpec((BN, R), lambda i: (i, 0)),
        ],
        out_shape=[
            jax.ShapeDtypeStruct((N, H), _f32),
            jax.ShapeDtypeStruct((N, R), _f32),
        ],
    )(x, s_nr, c_nr, U, V, g1, h0, bias1, ln1_g, ln1_b, w, bemb)


def _phase_d_body(hm_ref, agg_ref, ci_ref, rt_ref, w2_ref, b2_ref,
                  g_ref, bb_ref, out_ref):
    hm = hm_ref[...]
    ci = ci_ref[...]
    acc = jnp.dot(hm, rt_ref[...], preferred_element_type=_f32)
    acc = acc + b2_ref[...][None, :]
    for r in range(R):
        a_r = agg_ref[0, r] + agg_ref[1, r]
        acc = acc + jnp.dot(a_r * ci[:, r:r + 1], w2_ref[r],
                            preferred_element_type=_f32)
    m = jnp.mean(acc, axis=-1, keepdims=True)
    var = jnp.mean((acc - m) ** 2, axis=-1, keepdims=True)
    y = (acc - m) / jnp.sqrt(var + 1e-5) * g_ref[...][None, :] + bb_ref[...][None, :]
    out_ref[...] = jnp.maximum(y, 0.0) + hm


def _phase_d(hmid, agg, cinv, root2, rel_W2, bias2, ln2_g, ln2_b):
    full = lambda shp: pl.BlockSpec(shp, lambda i: tuple(0 for _ in shp))
    return pl.pallas_call(
        _phase_d_body,
        grid=(N // BN,),
        in_specs=[
            pl.BlockSpec((BN, H), lambda i: (i, 0)),
            pl.BlockSpec((NC, R, BN, H), lambda i: (0, 0, i, 0)),
            pl.BlockSpec((BN, R), lambda i: (i, 0)),
            full((H, H)), full((R, H, H)), full((H,)), full((H,)), full((H,)),
        ],
        out_specs=pl.BlockSpec((BN, H), lambda i: (i, 0)),
        out_shape=jax.ShapeDtypeStruct((N, H), _f32),
    )(hmid, agg, cinv, root2, rel_W2, bias2, ln2_g, ln2_b)


def kernel(x, edge_index, W_emb, b_emb, rp_W1, rp_b1, rp_W2, rp_b2,
           rel_W1, root1, bias1, ln1_g, ln1_b,
           rel_W2, root2, bias2, ln2_g, ln2_b):
    row = edge_index[0]
    col = edge_index[1]
    xf = x[:, 0]
    w = W_emb[0]

    ones_in = jnp.ones((BLK,), _f32)
    zeros_a = jnp.zeros((SLOT_PER_SUB,), _f32)
    zeros_c = jnp.zeros((AGG_PER_SUB, H), _f32)

    padrow = jnp.zeros((WAVE_E,), _i32)
    padcol = N + (jnp.arange(WAVE_E, dtype=_i32) % 128)

    xr, xc = _phase_a1(xf, row, col)
    et = _phase_a2(xr, xc, w, b_emb, rp_W1, rp_b1, rp_W2, rp_b2)
    rowp = jnp.concatenate([row, jnp.zeros((E2 - E,), _i32)])
    colp = jnp.concatenate([col, jnp.full((E2 - E,), N, _i32)])
    sum_p, cnt_p, crow, ccol, cnts = _phase_a3(
        colp, et, xr, rowp, ones_in, zeros_a, padrow, padcol)
    s_nr = sum_p[:, :N * R].reshape(NC, N, R)
    c_nr = cnt_p[:, :N * R].reshape(NC, N, R)
    crow5 = crow.reshape(NW, R, NWAVES, WAVE, G)
    ccol5 = ccol.reshape(NW, R, NWAVES, WAVE, G)

    hp = lax.Precision.HIGHEST
    U = jnp.einsum("h,rhk->rk", w, rel_W1, precision=hp)
    V = jnp.einsum("h,rhk->rk", b_emb, rel_W1, precision=hp)
    g1 = jnp.dot(w, root1, precision=hp)
    h0 = jnp.dot(b_emb, root1, precision=hp)

    hmid, cinv = _phase_b(x, s_nr, c_nr, U, V, g1, h0, bias1, ln1_g, ln1_b,
                          w, b_emb)
    agg = _phase_c(hmid, crow5, ccol5, cnts, zeros_c)
    return _phase_d(hmid, agg, cinv, root2, rel_W2, bias2, ln2_g, ln2_b)


# speedup vs baseline: 1.3049x; 1.3049x over previous
import functools

import jax
import jax.numpy as jnp
from jax import lax
from jax.experimental import pallas as pl
from jax.experimental.pallas import tpu as pltpu
from jax.experimental.pallas import tpu_sc as plsc

N = 50000
E = 800000
H = 32
R = 3

NC = 2
NS = 16
NW = NC * NS
G = 128
BLK = 1280
NGRP = BLK // G
NBLKS = E // BLK
BPW = -(-NBLKS // NW)
E2 = 819200
BE = 8192

NRP = 153600
SLOT_PER_SUB = NRP // NS
AGG_ROWS = 50176
AGG_PER_SUB = AGG_ROWS // NS
WAVE = 5

_mesh = plsc.VectorSubcoreMesh(core_axis_name="c", subcore_axis_name="s")
_f32 = jnp.float32
_i32 = jnp.int32


@functools.partial(
    pl.kernel,
    out_type=[
        jax.ShapeDtypeStruct((E2,), _f32),
        jax.ShapeDtypeStruct((E2,), _f32),
    ],
    mesh=_mesh,
    scratch_types=[
        pltpu.VMEM((BLK,), _i32),
        pltpu.VMEM((BLK,), _i32),
        pltpu.VMEM((BLK,), _f32),
        pltpu.VMEM((BLK,), _f32),
        pltpu.SemaphoreType.DMA,
    ],
)
def _phase_a1(x_hbm, row_hbm, col_hbm, xr_hbm, xc_hbm,
              rowb, colb, xrb, xcb, semg):
    cid = lax.axis_index("c")
    sid = lax.axis_index("s")
    wid = cid * NS + sid

    @pl.loop(0, BPW)
    def _block(bi):
        cidx = wid + bi * NW

        @pl.when(cidx < NBLKS)
        def _():
            ebase = cidx * BLK
            cl = pltpu.async_copy(row_hbm.at[pl.ds(ebase, BLK)], rowb, semg)
            cl2 = pltpu.async_copy(col_hbm.at[pl.ds(ebase, BLK)], colb, semg)
            cl.wait()
            cl2.wait()
            gs = []
            for i in range(NGRP):
                sl = pl.ds(i * G, G)
                gs.append(pltpu.async_copy(
                    x_hbm.at[rowb.at[sl]], xrb.at[sl], semg))
                gs.append(pltpu.async_copy(
                    x_hbm.at[colb.at[sl]], xcb.at[sl], semg))
            for c in gs:
                c.wait()
            st = [pltpu.async_copy(xrb, xr_hbm.at[pl.ds(ebase, BLK)], semg),
                  pltpu.async_copy(xcb, xc_hbm.at[pl.ds(ebase, BLK)], semg)]
            for c in st:
                c.wait()


def _phase_a2_body(xr_ref, xc_ref, wt_ref, bt_ref, w1_ref, b1_ref,
                   w2_ref, b2_ref, et_ref):
    xr = xr_ref[...]
    xc = xc_ref[...]
    wt = wt_ref[...]
    bt = bt_ref[...]
    efr = wt * xr[None, :] + bt
    efc = wt * xc[None, :] + bt
    ef = jnp.transpose(jnp.concatenate([efr, efc], axis=0))
    hid = jnp.maximum(jnp.dot(ef, w1_ref[...]) + b1_ref[...][None, :], 0.0)
    logits = jnp.dot(hid, w2_ref[...])
    lt = jnp.transpose(logits)
    b2 = b2_ref[...]
    l0 = lt[0] + b2[0]
    l1 = lt[1] + b2[1]
    l2 = lt[2] + b2[2]
    m = jnp.maximum(jnp.maximum(l0, l1), l2)
    u0 = jnp.exp(l0 - m)
    u1 = jnp.exp(l1 - m)
    u2 = jnp.exp(l2 - m)
    s = u0 + u1 + u2
    p0 = u0 / s
    p1 = u1 / s
    p2 = u2 / s
    et = jnp.zeros_like(et_ref[...])
    best = p0
    m1 = p1 > best
    et = jnp.where(m1, 1, et)
    best = jnp.where(m1, p1, best)
    et = jnp.where(p2 > best, 2, et)
    et_ref[...] = et


def _phase_a2(xr, xc, w, bemb, rp_W1, rp_b1, rp_W2, rp_b2):
    full = lambda shp: pl.BlockSpec(shp, lambda i: tuple(0 for _ in shp))
    wt = jnp.broadcast_to(w[:, None], (H, BE))
    bt = jnp.broadcast_to(bemb[:, None], (H, BE))
    return pl.pallas_call(
        _phase_a2_body,
        grid=(E2 // BE,),
        in_specs=[
            pl.BlockSpec((BE,), lambda i: (i,)),
            pl.BlockSpec((BE,), lambda i: (i,)),
            full((H, BE)), full((H, BE)), full((2 * H, H // 2)), full((H // 2,)),
            full((H // 2, R)), full((R,)),
        ],
        out_specs=pl.BlockSpec((BE,), lambda i: (i,)),
        out_shape=jax.ShapeDtypeStruct((E2,), _i32),
    )(xr, xc, wt, bt, rp_W1, rp_b1, rp_W2, rp_b2)


@functools.partial(
    pl.kernel,
    out_type=[
        jax.ShapeDtypeStruct((NC, NRP), _f32),
        jax.ShapeDtypeStruct((NC, NRP), _f32),
    ],
    mesh=_mesh,
    scratch_types=[
        pltpu.VMEM((BLK,), _i32),
        pltpu.VMEM((BLK,), _i32),
        pltpu.VMEM((BLK,), _f32),
        pltpu.VMEM((NGRP, G), _i32),
        pltpu.VMEM((BLK,), _f32),
        pltpu.VMEM_SHARED((NRP,), _f32),
        pltpu.VMEM_SHARED((NRP,), _f32),
        pltpu.SemaphoreType.DMA,
        pltpu.SemaphoreType.DMA,
    ],
)
def _phase_a3(col_hbm, et_hbm, xr_hbm, ones_hbm, zeros_hbm,
              sum_hbm, cnt_hbm,
              colb, etb, xrb, didx2, onesb, SUM, CNT, semg, sems):
    cid = lax.axis_index("c")
    sid = lax.axis_index("s")
    wid = cid * NS + sid

    pltpu.sync_copy(ones_hbm, onesb)
    base0 = sid * SLOT_PER_SUB
    pltpu.sync_copy(zeros_hbm, SUM.at[pl.ds(base0, SLOT_PER_SUB)])
    pltpu.sync_copy(zeros_hbm, CNT.at[pl.ds(base0, SLOT_PER_SUB)])
    plsc.subcore_barrier()

    @pl.loop(0, BPW)
    def _block(bi):
        cidx = wid + bi * NW

        @pl.when(cidx < NBLKS)
        def _():
            ebase = cidx * BLK
            ls = [pltpu.async_copy(col_hbm.at[pl.ds(ebase, BLK)], colb, semg),
                  pltpu.async_copy(et_hbm.at[pl.ds(ebase, BLK)], etb, semg),
                  pltpu.async_copy(xr_hbm.at[pl.ds(ebase, BLK)], xrb, semg)]
            for c in ls:
                c.wait()

            @pl.loop(0, NGRP)
            def _grp(gi):
                @pl.loop(0, G, step=16)
                def _compute(jj):
                    sj = pl.ds(gi * G + jj, 16)
                    didx2[gi, pl.ds(jj, 16)] = colb[sj] * R + etb[sj]

            ss = []
            for i in range(NGRP):
                sl = pl.ds(i * G, G)
                ss.append(pltpu.async_copy(
                    xrb.at[sl], SUM.at[didx2.at[i]], sems, add=True))
                ss.append(pltpu.async_copy(
                    onesb.at[sl], CNT.at[didx2.at[i]], sems, add=True))
            for c in ss:
                c.wait()

    plsc.subcore_barrier()
    sl = pl.ds(base0, SLOT_PER_SUB)
    pltpu.sync_copy(SUM.at[sl], sum_hbm.at[cid, sl])
    pltpu.sync_copy(CNT.at[sl], cnt_hbm.at[cid, sl])


@functools.partial(
    pl.kernel,
    out_type=jax.ShapeDtypeStruct((NC, R, AGG_ROWS, H), _f32),
    mesh=_mesh,
    scratch_types=[
        pltpu.VMEM((BLK,), _i32),
        pltpu.VMEM((BLK,), _i32),
        pltpu.VMEM((BLK,), _i32),
        pltpu.VMEM((NGRP, G), _i32),
        pltpu.VMEM((WAVE, G, H), _f32),
        pltpu.VMEM_SHARED((AGG_ROWS, H), _f32),
        pltpu.SemaphoreType.DMA,
        pltpu.SemaphoreType.DMA,
    ],
    compiler_params=pltpu.CompilerParams(use_tc_tiling_on_sc=False),
)
def _phase_c(hmid_hbm, row_hbm, col_hbm, et_hbm, zeros_hbm, agg_hbm,
             rowb, colb, etb, tgt2, grows, AGG, semg, sems):
    cid = lax.axis_index("c")
    sid = lax.axis_index("s")
    wid = cid * NS + sid
    rbase = sid * AGG_PER_SUB

    @pl.loop(0, R)
    def _rel(r):
        pltpu.sync_copy(zeros_hbm, AGG.at[pl.ds(rbase, AGG_PER_SUB), :])
        plsc.subcore_barrier()

        @pl.loop(0, BPW)
        def _block(bi):
            cidx = wid + bi * NW

            @pl.when(cidx < NBLKS)
            def _():
                ebase = cidx * BLK
                ls = [pltpu.async_copy(row_hbm.at[pl.ds(ebase, BLK)], rowb, semg),
                      pltpu.async_copy(col_hbm.at[pl.ds(ebase, BLK)], colb, semg),
                      pltpu.async_copy(et_hbm.at[pl.ds(ebase, BLK)], etb, semg)]
                for c in ls:
                    c.wait()

                @pl.loop(0, NGRP)
                def _grp(gi):
                    @pl.loop(0, G, step=16)
                    def _compute(jj):
                        j = gi * G + jj
                        sj = pl.ds(j, 16)
                        lane = jax.lax.iota(_i32, 16)
                        dump = N + lane + jj
                        tgt2[gi, pl.ds(jj, 16)] = jnp.where(
                            etb[sj] == r, colb[sj], dump)

                for w0 in range(0, NGRP, WAVE):
                    gs = [pltpu.async_copy(
                            hmid_hbm.at[rowb.at[pl.ds((w0 + i) * G, G)]],
                            grows.at[i], semg)
                          for i in range(WAVE)]
                    for c in gs:
                        c.wait()
                    ss = [pltpu.async_copy(
                            grows.at[i], AGG.at[tgt2.at[w0 + i]], sems, add=True)
                          for i in range(WAVE)]
                    for c in ss:
                        c.wait()

        plsc.subcore_barrier()
        pltpu.sync_copy(AGG.at[pl.ds(rbase, AGG_PER_SUB), :],
                        agg_hbm.at[cid, r, pl.ds(rbase, AGG_PER_SUB), :])
        plsc.subcore_barrier()


BN = 2000


def _phase_b_body(x_ref, s_ref, c_ref, u_ref, v_ref, g1_ref, h0_ref,
                  b1_ref, g_ref, bb_ref, w_ref, be_ref,
                  hmid_ref, cinv_ref):
    xb = x_ref[...]
    s = s_ref[0] + s_ref[1]
    c = c_ref[0] + c_ref[1]
    cc = jnp.maximum(c, 1.0)
    ci = 1.0 / cc
    msg = jnp.dot(s * ci, u_ref[...], preferred_element_type=_f32)
    msg = msg + jnp.dot(c * ci, v_ref[...], preferred_element_type=_f32)
    pre = xb * g1_ref[...][None, :] + h0_ref[...][None, :] + b1_ref[...][None, :] + msg
    m = jnp.mean(pre, axis=-1, keepdims=True)
    var = jnp.mean((pre - m) ** 2, axis=-1, keepdims=True)
    y = (pre - m) / jnp.sqrt(var + 1e-5) * g_ref[...][None, :] + bb_ref[...][None, :]
    h = xb * w_ref[...][None, :] + be_ref[...][None, :]
    hmid_ref[...] = jnp.maximum(y, 0.0) + h
    cinv_ref[...] = ci


def _phase_b(x, s_nr, c_nr, U, V, g1, h0, bias1, ln1_g, ln1_b, w, bemb):
    full = lambda shp: pl.BlockSpec(shp, lambda i: tuple(0 for _ in shp))
    return pl.pallas_call(
        _phase_b_body,
        grid=(N // BN,),
        in_specs=[
            pl.BlockSpec((BN, 1), lambda i: (i, 0)),
            pl.BlockSpec((NC, BN, R), lambda i: (0, i, 0)),
            pl.BlockSpec((NC, BN, R), lambda i: (0, i, 0)),
            full((R, H)), full((R, H)), full((H,)), full((H,)),
            full((H,)), full((H,)), full((H,)), full((H,)), full((H,)),
        ],
        out_specs=[
            pl.BlockSpec((BN, H), lambda i: (i, 0)),
            pl.BlockSpec((BN, R), lambda i: (i, 0)),
        ],
        out_shape=[
            jax.ShapeDtypeStruct((N, H), _f32),
            jax.ShapeDtypeStruct((N, R), _f32),
        ],
    )(x, s_nr, c_nr, U, V, g1, h0, bias1, ln1_g, ln1_b, w, bemb)


def _phase_d_body(hm_ref, agg_ref, ci_ref, rt_ref, w2_ref, b2_ref,
                  g_ref, bb_ref, out_ref):
    hm = hm_ref[...]
    ci = ci_ref[...]
    acc = jnp.dot(hm, rt_ref[...], preferred_element_type=_f32)
    acc = acc + b2_ref[...][None, :]
    for r in range(R):
        a_r = agg_ref[0, r] + agg_ref[1, r]
        acc = acc + jnp.dot(a_r * ci[:, r:r + 1], w2_ref[r],
                            preferred_element_type=_f32)
    m = jnp.mean(acc, axis=-1, keepdims=True)
    var = jnp.mean((acc - m) ** 2, axis=-1, keepdims=True)
    y = (acc - m) / jnp.sqrt(var + 1e-5) * g_ref[...][None, :] + bb_ref[...][None, :]
    out_ref[...] = jnp.maximum(y, 0.0) + hm


def _phase_d(hmid, agg, cinv, root2, rel_W2, bias2, ln2_g, ln2_b):
    full = lambda shp: pl.BlockSpec(shp, lambda i: tuple(0 for _ in shp))
    return pl.pallas_call(
        _phase_d_body,
        grid=(N // BN,),
        in_specs=[
            pl.BlockSpec((BN, H), lambda i: (i, 0)),
            pl.BlockSpec((NC, R, BN, H), lambda i: (0, 0, i, 0)),
            pl.BlockSpec((BN, R), lambda i: (i, 0)),
            full((H, H)), full((R, H, H)), full((H,)), full((H,)), full((H,)),
        ],
        out_specs=pl.BlockSpec((BN, H), lambda i: (i, 0)),
        out_shape=jax.ShapeDtypeStruct((N, H), _f32),
    )(hmid, agg, cinv, root2, rel_W2, bias2, ln2_g, ln2_b)


def kernel(x, edge_index, W_emb, b_emb, rp_W1, rp_b1, rp_W2, rp_b2,
           rel_W1, root1, bias1, ln1_g, ln1_b,
           rel_W2, root2, bias2, ln2_g, ln2_b):
    row = edge_index[0]
    col = edge_index[1]
    xf = x[:, 0]
    w = W_emb[0]

    ones_in = jnp.ones((BLK,), _f32)
    zeros_a = jnp.zeros((SLOT_PER_SUB,), _f32)
    zeros_c = jnp.zeros((AGG_PER_SUB, H), _f32)

    xr, xc = _phase_a1(xf, row, col)
    et = _phase_a2(xr, xc, w, b_emb, rp_W1, rp_b1, rp_W2, rp_b2)
    sum_p, cnt_p = _phase_a3(col, et, xr, ones_in, zeros_a)
    s_nr = sum_p[:, :N * R].reshape(NC, N, R)
    c_nr = cnt_p[:, :N * R].reshape(NC, N, R)

    hp = lax.Precision.HIGHEST
    U = jnp.einsum("h,rhk->rk", w, rel_W1, precision=hp)
    V = jnp.einsum("h,rhk->rk", b_emb, rel_W1, precision=hp)
    g1 = jnp.dot(w, root1, precision=hp)
    h0 = jnp.dot(b_emb, root1, precision=hp)

    hmid, cinv = _phase_b(x, s_nr, c_nr, U, V, g1, h0, bias1, ln1_g, ln1_b,
                          w, b_emb)
    agg = _phase_c(hmid, row, col, et, zeros_c)
    return _phase_d(hmid, agg, cinv, root2, rel_W2, bias2, ln2_g, ln2_b)
